# Initial kernel scaffold; baseline (speedup 1.0000x reference)
import jax
import jax.numpy as jnp
from jax.experimental import pallas as pl


def kernel(atoms, pos, edge_index, params):
    raise NotImplementedError("write your pallas kernel here")


# speedup vs baseline: 3.2249x; 3.2249x over previous
import functools

import jax
import jax.numpy as jnp
from jax import lax
from jax.experimental import pallas as pl
from jax.experimental.pallas import tpu as pltpu
from jax.experimental.pallas import tpu_sc as plsc

D = 128
LANES = 16
CHUNK = 128
NW = 32


def _ln(x, g, b):
    m = jnp.mean(x, axis=-1, keepdims=True)
    v = jnp.mean((x - m) ** 2, axis=-1, keepdims=True)
    return (x - m) / jnp.sqrt(v + 1e-5) * g + b


def _init_body(atoms_ref, emb_ref, w1a_ref, w1b_ref, b1_ref,
               h_ref, a_ref, b_ref):
    at = atoms_ref[...]
    lanes = lax.broadcasted_iota(jnp.int32, (1, D), 1)
    oh = (at == lanes).astype(jnp.float32)
    h = jnp.dot(oh, emb_ref[...], preferred_element_type=jnp.float32)
    h_ref[...] = h
    a_ref[...] = jnp.dot(h, w1a_ref[...], preferred_element_type=jnp.float32) + b1_ref[...]
    b_ref[...] = jnp.dot(h, w1b_ref[...], preferred_element_type=jnp.float32)


def _tc_init(atoms, emb_pad, w1a, w1b, b1, n, bn):
    grid = (n // bn,)
    return pl.pallas_call(
        _init_body,
        grid=grid,
        in_specs=[
            pl.BlockSpec((bn, 1), lambda i: (i, 0)),
            pl.BlockSpec((D, D), lambda i: (0, 0)),
            pl.BlockSpec((D, D), lambda i: (0, 0)),
            pl.BlockSpec((D, D), lambda i: (0, 0)),
            pl.BlockSpec((1, D), lambda i: (0, 0)),
        ],
        out_specs=[
            pl.BlockSpec((bn, D), lambda i: (i, 0)),
            pl.BlockSpec((bn, D), lambda i: (i, 0)),
            pl.BlockSpec((bn, D), lambda i: (i, 0)),
        ],
        out_shape=[
            jax.ShapeDtypeStruct((n, D), jnp.float32),
            jax.ShapeDtypeStruct((n, D), jnp.float32),
            jax.ShapeDtypeStruct((n, D), jnp.float32),
        ],
    )(atoms, emb_pad, w1a, w1b, b1)


def _make_sc_gather(n, e):
    nchunks = e // CHUNK
    base_c = nchunks // NW
    extra_c = nchunks % NW
    mesh = plsc.VectorSubcoreMesh(core_axis_name="c", subcore_axis_name="s")

    @functools.partial(
        pl.kernel,
        mesh=mesh,
        out_type=[
            jax.ShapeDtypeStruct((e, D), jnp.float32),
            jax.ShapeDtypeStruct((3, e), jnp.float32),
        ],
        scratch_types=[
            pltpu.VMEM((CHUNK,), jnp.int32),
            pltpu.VMEM((CHUNK,), jnp.int32),
            pltpu.VMEM((CHUNK, D), jnp.float32),
            pltpu.VMEM((CHUNK, D), jnp.float32),
            pltpu.VMEM((CHUNK,), jnp.float32),
            pltpu.VMEM((CHUNK,), jnp.float32),
            pltpu.VMEM((CHUNK,), jnp.float32),
            pltpu.VMEM((CHUNK,), jnp.float32),
            pltpu.VMEM((CHUNK,), jnp.float32),
            pltpu.VMEM((CHUNK,), jnp.float32),
            pltpu.VMEM((3, CHUNK), jnp.float32),
            pltpu.SemaphoreType.DMA,
            pltpu.SemaphoreType.DMA,
            pltpu.SemaphoreType.DMA,
        ],
    )
    def gather_k(a_hbm, b_hbm, px_hbm, py_hbm, pz_hbm, src_hbm, dst_hbm,
                 msg1p_hbm, diff_hbm,
                 idx_d, idx_s, rows_a, rows_b,
                 gxi, gyi, gzi, gxj, gyj, gzj, dxyz,
                 sem_a, sem_b, sem_p):
        cid = lax.axis_index("c")
        sid = lax.axis_index("s")
        wid = sid * 2 + cid
        nloc = base_c + jnp.where(wid < extra_c, 1, 0)
        start = wid * base_c + jnp.minimum(wid, extra_c)

        def chunk_body(j, carry):
            c = start + j
            eb = c * CHUNK
            pltpu.sync_copy(dst_hbm.at[pl.ds(eb, CHUNK)], idx_d)
            pltpu.sync_copy(src_hbm.at[pl.ds(eb, CHUNK)], idx_s)
            cpa = pltpu.async_copy(a_hbm.at[idx_d], rows_a, sem_a)
            cpb = pltpu.async_copy(b_hbm.at[idx_s], rows_b, sem_b)
            cp1 = pltpu.async_copy(px_hbm.at[idx_d], gxi, sem_p)
            cp2 = pltpu.async_copy(py_hbm.at[idx_d], gyi, sem_p)
            cp3 = pltpu.async_copy(pz_hbm.at[idx_d], gzi, sem_p)
            cp4 = pltpu.async_copy(px_hbm.at[idx_s], gxj, sem_p)
            cp5 = pltpu.async_copy(py_hbm.at[idx_s], gyj, sem_p)
            cp6 = pltpu.async_copy(pz_hbm.at[idx_s], gzj, sem_p)
            cp1.wait()
            cp2.wait()
            cp3.wait()
            cp4.wait()
            cp5.wait()
            cp6.wait()
            for g in range(CHUNK // LANES):
                sl = pl.ds(g * LANES, LANES)
                dxyz[0, sl] = gxi[sl] - gxj[sl]
                dxyz[1, sl] = gyi[sl] - gyj[sl]
                dxyz[2, sl] = gzi[sl] - gzj[sl]
            pltpu.sync_copy(dxyz, diff_hbm.at[:, pl.ds(eb, CHUNK)])

            cpa.wait()
            cpb.wait()

            def row_body(r, carry2):
                for u in range(D // LANES):
                    sl = pl.ds(u * LANES, LANES)
                    rows_a[r, sl] = rows_a[r, sl] + rows_b[r, sl]
                return carry2

            lax.fori_loop(0, CHUNK, row_body, 0)
            pltpu.sync_copy(rows_a, msg1p_hbm.at[pl.ds(eb, CHUNK)])
            return carry

        lax.fori_loop(0, nloc, chunk_body, 0)

    return gather_k


def _edge_body(msg1p_ref, diff_ref, wd_ref,
               g1_ref, c1_ref, w2_ref, b2_ref, g2_ref, c2_ref,
               wp1_ref, bp1_ref, gp_ref, cp_ref, wp2_ref, bp2_ref,
               msg_ref, pw_ref):
    dT = jnp.transpose(diff_ref[...])
    dist = jnp.sqrt(jnp.sum(dT * dT, axis=-1, keepdims=True))
    x = msg1p_ref[...] + dist * wd_ref[...]
    x = jnp.maximum(_ln(x, g1_ref[...], c1_ref[...]), 0.0)
    x = jnp.dot(x, w2_ref[...], preferred_element_type=jnp.float32) + b2_ref[...]
    msg = jnp.maximum(_ln(x, g2_ref[...], c2_ref[...]), 0.0)
    msg_ref[...] = msg
    p = jnp.dot(msg, wp1_ref[...], preferred_element_type=jnp.float32) + bp1_ref[...]
    p = jnp.maximum(_ln(p, gp_ref[...], cp_ref[...]), 0.0)
    pw = jnp.sum(p * wp2_ref[...], axis=-1) + bp2_ref[0, 0]
    pw_ref[...] = pw.reshape(1, -1)


def _tc_edge(msg1p, diff, wts, e, be):
    grid = (e // be,)
    full = lambda i: (0, 0)
    return pl.pallas_call(
        _edge_body,
        grid=grid,
        in_specs=[
            pl.BlockSpec((be, D), lambda i: (i, 0)),
            pl.BlockSpec((3, be), lambda i: (0, i)),
        ] + [pl.BlockSpec(w.shape, full) for w in wts],
        out_specs=[
            pl.BlockSpec((be, D), lambda i: (i, 0)),
            pl.BlockSpec((1, be), lambda i: (0, i)),
        ],
        out_shape=[
            jax.ShapeDtypeStruct((e, D), jnp.float32),
            jax.ShapeDtypeStruct((1, e), jnp.float32),
        ],
    )(msg1p, diff, *wts)


def _make_sc_scatter(np_, e):
    nchunks = e // CHUNK
    base_c = nchunks // NW
    extra_c = nchunks % NW
    rows_per_tile = np_ // 16
    mesh = plsc.VectorSubcoreMesh(core_axis_name="c", subcore_axis_name="s")

    @functools.partial(
        pl.kernel,
        mesh=mesh,
        out_type=[
            jax.ShapeDtypeStruct((2, np_, D), jnp.float32),
            jax.ShapeDtypeStruct((2 * 4 * np_,), jnp.float32),
        ],
        scratch_types=[
            pltpu.VMEM((CHUNK,), jnp.int32),
            pltpu.VMEM((CHUNK, D), jnp.float32),
            pltpu.VMEM((1, CHUNK), jnp.float32),
            pltpu.VMEM((3, CHUNK), jnp.float32),
            pltpu.VMEM((CHUNK,), jnp.float32),
            pltpu.VMEM((CHUNK,), jnp.float32),
            pltpu.VMEM((CHUNK,), jnp.float32),
            pltpu.VMEM((CHUNK,), jnp.float32),
            pltpu.VMEM((CHUNK, D), jnp.float32),
            pltpu.VMEM((rows_per_tile,), jnp.float32),
            pltpu.VMEM_SHARED((np_, D), jnp.float32),
            pltpu.VMEM_SHARED((np_,), jnp.float32),
            pltpu.VMEM_SHARED((np_,), jnp.float32),
            pltpu.VMEM_SHARED((np_,), jnp.float32),
            pltpu.VMEM_SHARED((np_,), jnp.float32),
        ],
    )
    def scatter_k(msg_hbm, pw_hbm, diff_hbm, dst_hbm,
                  magg_hbm, pagg_hbm,
                  idx_d, rows_v, pw_v, diff_v, vx_v, vy_v, vz_v, ones_v,
                  z_v, z1_v, acc_msg, acc_x, acc_y, acc_z, acc_n):
        cid = lax.axis_index("c")
        sid = lax.axis_index("s")
        wid = sid * 2 + cid
        nloc = base_c + jnp.where(wid < extra_c, 1, 0)
        start = wid * base_c + jnp.minimum(wid, extra_c)

        zero16 = jnp.zeros((LANES,), jnp.float32)
        one16 = jnp.ones((LANES,), jnp.float32)

        def fill_body(r, carry):
            for u in range(D // LANES):
                z_v[r, pl.ds(u * LANES, LANES)] = zero16
            return carry

        lax.fori_loop(0, CHUNK, fill_body, 0)

        def fill1_body(r, carry):
            z1_v[pl.ds(r * LANES, LANES)] = zero16
            return carry

        lax.fori_loop(0, rows_per_tile // LANES, fill1_body, 0)
        for g in range(CHUNK // LANES):
            ones_v[pl.ds(g * LANES, LANES)] = one16

        rbase = sid * rows_per_tile
        for k in range(rows_per_tile // CHUNK):
            pltpu.sync_copy(z_v, acc_msg.at[pl.ds(rbase + k * CHUNK, CHUNK)])
        pltpu.sync_copy(z1_v, acc_x.at[pl.ds(rbase, rows_per_tile)])
        pltpu.sync_copy(z1_v, acc_y.at[pl.ds(rbase, rows_per_tile)])
        pltpu.sync_copy(z1_v, acc_z.at[pl.ds(rbase, rows_per_tile)])
        pltpu.sync_copy(z1_v, acc_n.at[pl.ds(rbase, rows_per_tile)])
        plsc.subcore_barrier()

        def chunk_body(j, carry):
            c = start + j
            eb = c * CHUNK
            pltpu.sync_copy(dst_hbm.at[pl.ds(eb, CHUNK)], idx_d)
            pltpu.sync_copy(msg_hbm.at[pl.ds(eb, CHUNK)], rows_v)
            pltpu.sync_copy(pw_hbm.at[:, pl.ds(eb, CHUNK)], pw_v)
            pltpu.sync_copy(diff_hbm.at[:, pl.ds(eb, CHUNK)], diff_v)
            pltpu.sync_copy(rows_v, acc_msg.at[idx_d], add=True)
            for g in range(CHUNK // LANES):
                sl = pl.ds(g * LANES, LANES)
                pwg = pw_v[0, sl]
                vx_v[sl] = diff_v[0, sl] * pwg
                vy_v[sl] = diff_v[1, sl] * pwg
                vz_v[sl] = diff_v[2, sl] * pwg
            pltpu.sync_copy(vx_v, acc_x.at[idx_d], add=True)
            pltpu.sync_copy(vy_v, acc_y.at[idx_d], add=True)
            pltpu.sync_copy(vz_v, acc_z.at[idx_d], add=True)
            pltpu.sync_copy(ones_v, acc_n.at[idx_d], add=True)
            return carry

        lax.fori_loop(0, nloc, chunk_body, 0)
        plsc.subcore_barrier()
        pltpu.sync_copy(acc_msg.at[pl.ds(rbase, rows_per_tile)],
                        magg_hbm.at[cid, pl.ds(rbase, rows_per_tile)])
        for cc, acc in enumerate([acc_x, acc_y, acc_z, acc_n]):
            pltpu.sync_copy(acc.at[pl.ds(rbase, rows_per_tile)],
                            pagg_hbm.at[pl.ds((cid * 4 + cc) * np_ + rbase,
                                              rows_per_tile)])

    return scatter_k


def _node_body(h_ref, pos_ref, magg_ref, pagg_ref,
               wu1h_ref, wu1m_ref, bu1_ref, gu1_ref, cu1_ref,
               wu2_ref, bu2_ref, gu2_ref, cu2_ref,
               w1a_ref, w1b_ref, b1_ref,
               hn_ref, posn_ref, a_ref, b_ref):
    h = h_ref[...]
    magg = magg_ref[0] + magg_ref[1]
    pagg = pagg_ref[0] + pagg_ref[1]
    lanes = lax.broadcasted_iota(jnp.int32, (1, 4), 1)
    cnt = jnp.sum(pagg * (lanes == 3).astype(jnp.float32), axis=-1, keepdims=True)
    cnt = jnp.maximum(cnt, 1.0)
    posd = pagg * (lanes < 3).astype(jnp.float32) / cnt
    posn_ref[...] = pos_ref[...] + posd

    u = (jnp.dot(h, wu1h_ref[...], preferred_element_type=jnp.float32)
         + jnp.dot(magg, wu1m_ref[...], preferred_element_type=jnp.float32)
         + bu1_ref[...])
    u = jnp.maximum(_ln(u, gu1_ref[...], cu1_ref[...]), 0.0)
    u = jnp.dot(u, wu2_ref[...], preferred_element_type=jnp.float32) + bu2_ref[...]
    u = jnp.maximum(_ln(u, gu2_ref[...], cu2_ref[...]), 0.0)
    hn = h + u
    hn_ref[...] = hn
    a_ref[...] = jnp.dot(hn, w1a_ref[...], preferred_element_type=jnp.float32) + b1_ref[...]
    b_ref[...] = jnp.dot(hn, w1b_ref[...], preferred_element_type=jnp.float32)


def _tc_node(h, pos4, magg_p, pagg_p, wts, n, bn):
    grid = (n // bn,)
    full = lambda i: (0, 0)
    return pl.pallas_call(
        _node_body,
        grid=grid,
        in_specs=[
            pl.BlockSpec((bn, D), lambda i: (i, 0)),
            pl.BlockSpec((bn, 4), lambda i: (i, 0)),
            pl.BlockSpec((2, bn, D), lambda i: (0, i, 0)),
            pl.BlockSpec((2, bn, 4), lambda i: (0, i, 0)),
        ] + [pl.BlockSpec(w.shape, full) for w in wts],
        out_specs=[
            pl.BlockSpec((bn, D), lambda i: (i, 0)),
            pl.BlockSpec((bn, 4), lambda i: (i, 0)),
            pl.BlockSpec((bn, D), lambda i: (i, 0)),
            pl.BlockSpec((bn, D), lambda i: (i, 0)),
        ],
        out_shape=[
            jax.ShapeDtypeStruct((n, D), jnp.float32),
            jax.ShapeDtypeStruct((n, 4), jnp.float32),
            jax.ShapeDtypeStruct((n, D), jnp.float32),
            jax.ShapeDtypeStruct((n, D), jnp.float32),
        ],
    )(h, pos4, magg_p, pagg_p, *wts)


def kernel(atoms, pos, edge_index, params):
    n = atoms.shape[0]
    e = edge_index.shape[1]
    layers = params["layers"]

    emb = params["embedding"]
    emb_pad = jnp.zeros((D, D), jnp.float32).at[: emb.shape[0]].set(emb)

    def msg1_parts(lp):
        w1 = lp["msg_l1"]["W"]
        return (w1[:D], w1[D:2 * D], w1[2 * D:2 * D + 1],
                lp["msg_l1"]["b"].reshape(1, D))

    src = edge_index[0].astype(jnp.int32)
    dst = edge_index[1].astype(jnp.int32)
    posf = pos.astype(jnp.float32)
    pos4 = jnp.pad(posf, ((0, 0), (0, 4 - pos.shape[1])))

    bn = 2000
    be = 2560
    w1a0, w1b0, _, b10 = msg1_parts(layers[0])
    h, a_cur, b_cur = _tc_init(atoms.astype(jnp.int32), emb_pad,
                               w1a0, w1b0, b10, n, bn)

    np_ = ((n + NW * 8 - 1) // (NW * 8)) * NW * 8
    gather_k = _make_sc_gather(n, e)
    scatter_k = _make_sc_scatter(np_, e)

    for li, lp in enumerate(layers):
        px, py, pz = pos4[:, 0], pos4[:, 1], pos4[:, 2]
        msg1p, diff3 = gather_k(a_cur, b_cur, px, py, pz, src, dst)

        wd = msg1_parts(lp)[2]
        ewts = [
            wd,
            lp["msg_ln1"]["g"].reshape(1, D), lp["msg_ln1"]["b"].reshape(1, D),
            lp["msg_l2"]["W"], lp["msg_l2"]["b"].reshape(1, D),
            lp["msg_ln2"]["g"].reshape(1, D), lp["msg_ln2"]["b"].reshape(1, D),
            lp["pos_l1"]["W"], lp["pos_l1"]["b"].reshape(1, D),
            lp["pos_ln1"]["g"].reshape(1, D), lp["pos_ln1"]["b"].reshape(1, D),
            lp["pos_l2"]["W"].reshape(1, D), lp["pos_l2"]["b"].reshape(1, 1),
        ]
        msg, pw = _tc_edge(msg1p, diff3, ewts, e, be)

        magg_p, pagg_flat = scatter_k(msg, pw, diff3, dst)
        pagg_p = pagg_flat.reshape(2, 4, np_).transpose(0, 2, 1)

        nxt = layers[li + 1] if li + 1 < len(layers) else layers[0]
        w1a_n, w1b_n, _, b1_n = msg1_parts(nxt)
        wu1 = lp["upd_l1"]["W"]
        nwts = [
            wu1[:D], wu1[D:], lp["upd_l1"]["b"].reshape(1, D),
            lp["upd_ln1"]["g"].reshape(1, D), lp["upd_ln1"]["b"].reshape(1, D),
            lp["upd_l2"]["W"], lp["upd_l2"]["b"].reshape(1, D),
            lp["upd_ln2"]["g"].reshape(1, D), lp["upd_ln2"]["b"].reshape(1, D),
            w1a_n, w1b_n, b1_n,
        ]
        h, pos4, a_cur, b_cur = _tc_node(h, pos4, magg_p, pagg_p, nwts, n, bn)

    return pos4[:, :3]


# speedup vs baseline: 4.0333x; 1.2507x over previous
import functools

import jax
import jax.numpy as jnp
from jax import lax
from jax.experimental import pallas as pl
from jax.experimental.pallas import tpu as pltpu
from jax.experimental.pallas import tpu_sc as plsc

D = 128
LANES = 16
CHUNK = 128
NW = 32


def _ln(x, g, b):
    m = jnp.mean(x, axis=-1, keepdims=True)
    v = jnp.mean((x - m) ** 2, axis=-1, keepdims=True)
    return (x - m) / jnp.sqrt(v + 1e-5) * g + b


def _init_body(atoms_ref, emb_ref, w1a_ref, w1b_ref, b1_ref,
               h_ref, a_ref, b_ref):
    at = atoms_ref[...]
    lanes = lax.broadcasted_iota(jnp.int32, (1, D), 1)
    oh = (at == lanes).astype(jnp.float32)
    h = jnp.dot(oh, emb_ref[...], preferred_element_type=jnp.float32)
    h_ref[...] = h
    a_ref[...] = jnp.dot(h, w1a_ref[...], preferred_element_type=jnp.float32) + b1_ref[...]
    b_ref[...] = jnp.dot(h, w1b_ref[...], preferred_element_type=jnp.float32)


def _tc_init(atoms, emb_pad, w1a, w1b, b1, n, bn):
    grid = (n // bn,)
    return pl.pallas_call(
        _init_body,
        grid=grid,
        in_specs=[
            pl.BlockSpec((bn, 1), lambda i: (i, 0)),
            pl.BlockSpec((D, D), lambda i: (0, 0)),
            pl.BlockSpec((D, D), lambda i: (0, 0)),
            pl.BlockSpec((D, D), lambda i: (0, 0)),
            pl.BlockSpec((1, D), lambda i: (0, 0)),
        ],
        out_specs=[
            pl.BlockSpec((bn, D), lambda i: (i, 0)),
            pl.BlockSpec((bn, D), lambda i: (i, 0)),
            pl.BlockSpec((bn, D), lambda i: (i, 0)),
        ],
        out_shape=[
            jax.ShapeDtypeStruct((n, D), jnp.float32),
            jax.ShapeDtypeStruct((n, D), jnp.float32),
            jax.ShapeDtypeStruct((n, D), jnp.float32),
        ],
    )(atoms, emb_pad, w1a, w1b, b1)


def _make_sc_gather(n, e):
    npairs = e // (2 * CHUNK)
    base_p = npairs // NW
    extra_p = npairs % NW
    mesh = plsc.VectorSubcoreMesh(core_axis_name="c", subcore_axis_name="s")

    buf_types = [
        pltpu.VMEM((CHUNK,), jnp.int32),
        pltpu.VMEM((CHUNK,), jnp.int32),
        pltpu.VMEM((CHUNK, D), jnp.float32),
        pltpu.VMEM((CHUNK, D), jnp.float32),
        pltpu.VMEM((CHUNK,), jnp.float32),
        pltpu.VMEM((CHUNK,), jnp.float32),
        pltpu.VMEM((CHUNK,), jnp.float32),
        pltpu.VMEM((CHUNK,), jnp.float32),
        pltpu.VMEM((CHUNK,), jnp.float32),
        pltpu.VMEM((CHUNK,), jnp.float32),
        pltpu.VMEM((3, CHUNK), jnp.float32),
        pltpu.SemaphoreType.DMA,
        pltpu.SemaphoreType.DMA,
        pltpu.SemaphoreType.DMA,
    ]

    @functools.partial(
        pl.kernel,
        mesh=mesh,
        out_type=[
            jax.ShapeDtypeStruct((e, D), jnp.float32),
            jax.ShapeDtypeStruct((3, e), jnp.float32),
        ],
        scratch_types=buf_types + buf_types + [pltpu.SemaphoreType.DMA],
    )
    def gather_k(a_hbm, b_hbm, px_hbm, py_hbm, pz_hbm, src_hbm, dst_hbm,
                 msg1p_hbm, diff_hbm, *scratch):
        bufs = (scratch[:14], scratch[14:28])
        sem_i = scratch[28]
        cid = lax.axis_index("c")
        sid = lax.axis_index("s")
        wid = sid * 2 + cid
        nloc = base_p + jnp.where(wid < extra_p, 1, 0)
        pstart = wid * base_p + jnp.minimum(wid, extra_p)

        def drain_writes(b):
            (idx_d, idx_s, rows_a, rows_b, gxi, gyi, gzi, gxj, gyj, gzj,
             dxyz, sem_g, sem_p, sem_w) = bufs[b]
            pltpu.make_async_copy(rows_a, msg1p_hbm.at[pl.ds(0, CHUNK)],
                                  sem_w).wait()
            pltpu.make_async_copy(dxyz, diff_hbm.at[:, pl.ds(0, CHUNK)],
                                  sem_w).wait()

        def pair_body(g, carry):
            eb0 = (pstart + g) * 2 * CHUNK
            idx_cps = []
            for b in (0, 1):
                idx_d, idx_s = bufs[b][0], bufs[b][1]
                eb = eb0 + b * CHUNK
                idx_cps.append(pltpu.async_copy(
                    dst_hbm.at[pl.ds(eb, CHUNK)], idx_d, sem_i))
                idx_cps.append(pltpu.async_copy(
                    src_hbm.at[pl.ds(eb, CHUNK)], idx_s, sem_i))
            for cp in idx_cps:
                cp.wait()

            row_cps = []
            pos_cps = []
            for b in (0, 1):
                (idx_d, idx_s, rows_a, rows_b, gxi, gyi, gzi, gxj, gyj, gzj,
                 dxyz, sem_g, sem_p, sem_w) = bufs[b]

                @pl.when(g > 0)
                def _():
                    drain_writes(b)

                row_cps.append((pltpu.async_copy(a_hbm.at[idx_d], rows_a, sem_g),
                                pltpu.async_copy(b_hbm.at[idx_s], rows_b, sem_g)))
                pos_cps.append([
                    pltpu.async_copy(px_hbm.at[idx_d], gxi, sem_p),
                    pltpu.async_copy(py_hbm.at[idx_d], gyi, sem_p),
                    pltpu.async_copy(pz_hbm.at[idx_d], gzi, sem_p),
                    pltpu.async_copy(px_hbm.at[idx_s], gxj, sem_p),
                    pltpu.async_copy(py_hbm.at[idx_s], gyj, sem_p),
                    pltpu.async_copy(pz_hbm.at[idx_s], gzj, sem_p),
                ])

            for b in (0, 1):
                (idx_d, idx_s, rows_a, rows_b, gxi, gyi, gzi, gxj, gyj, gzj,
                 dxyz, sem_g, sem_p, sem_w) = bufs[b]
                eb = eb0 + b * CHUNK
                for cp in pos_cps[b]:
                    cp.wait()
                for u in range(CHUNK // LANES):
                    sl = pl.ds(u * LANES, LANES)
                    dxyz[0, sl] = gxi[sl] - gxj[sl]
                    dxyz[1, sl] = gyi[sl] - gyj[sl]
                    dxyz[2, sl] = gzi[sl] - gzj[sl]
                pltpu.async_copy(dxyz, diff_hbm.at[:, pl.ds(eb, CHUNK)], sem_w)
                row_cps[b][0].wait()
                row_cps[b][1].wait()

                def row_body(r, carry2):
                    for u in range(D // LANES):
                        sl = pl.ds(u * LANES, LANES)
                        rows_a[r, sl] = rows_a[r, sl] + rows_b[r, sl]
                    return carry2

                lax.fori_loop(0, CHUNK, row_body, 0)
                pltpu.async_copy(rows_a, msg1p_hbm.at[pl.ds(eb, CHUNK)], sem_w)
            return carry

        lax.fori_loop(0, nloc, pair_body, 0)
        drain_writes(0)
        drain_writes(1)

    return gather_k


def _edge_body(msg1p_ref, diff_ref, wd_ref,
               g1_ref, c1_ref, w2_ref, b2_ref, g2_ref, c2_ref,
               wp1_ref, bp1_ref, gp_ref, cp_ref, wp2_ref, bp2_ref,
               msg_ref, pw_ref):
    dT = jnp.transpose(diff_ref[...])
    dist = jnp.sqrt(jnp.sum(dT * dT, axis=-1, keepdims=True))
    x = msg1p_ref[...] + dist * wd_ref[...]
    x = jnp.maximum(_ln(x, g1_ref[...], c1_ref[...]), 0.0)
    x = jnp.dot(x, w2_ref[...], preferred_element_type=jnp.float32) + b2_ref[...]
    msg = jnp.maximum(_ln(x, g2_ref[...], c2_ref[...]), 0.0)
    msg_ref[...] = msg
    p = jnp.dot(msg, wp1_ref[...], preferred_element_type=jnp.float32) + bp1_ref[...]
    p = jnp.maximum(_ln(p, gp_ref[...], cp_ref[...]), 0.0)
    pw = jnp.sum(p * wp2_ref[...], axis=-1) + bp2_ref[0, 0]
    pw_ref[...] = pw.reshape(1, -1)


def _tc_edge(msg1p, diff, wts, e, be):
    grid = (e // be,)
    full = lambda i: (0, 0)
    return pl.pallas_call(
        _edge_body,
        grid=grid,
        in_specs=[
            pl.BlockSpec((be, D), lambda i: (i, 0)),
            pl.BlockSpec((3, be), lambda i: (0, i)),
        ] + [pl.BlockSpec(w.shape, full) for w in wts],
        out_specs=[
            pl.BlockSpec((be, D), lambda i: (i, 0)),
            pl.BlockSpec((1, be), lambda i: (0, i)),
        ],
        out_shape=[
            jax.ShapeDtypeStruct((e, D), jnp.float32),
            jax.ShapeDtypeStruct((1, e), jnp.float32),
        ],
    )(msg1p, diff, *wts)


def _make_sc_scatter(np_, e):
    npairs = e // (2 * CHUNK)
    base_p = npairs // NW
    extra_p = npairs % NW
    rows_per_tile = np_ // 16
    mesh = plsc.VectorSubcoreMesh(core_axis_name="c", subcore_axis_name="s")

    buf_types = [
        pltpu.VMEM((CHUNK,), jnp.int32),
        pltpu.VMEM((CHUNK, D), jnp.float32),
        pltpu.VMEM((1, CHUNK), jnp.float32),
        pltpu.VMEM((3, CHUNK), jnp.float32),
        pltpu.VMEM((CHUNK,), jnp.float32),
        pltpu.VMEM((CHUNK,), jnp.float32),
        pltpu.VMEM((CHUNK,), jnp.float32),
        pltpu.SemaphoreType.DMA,
        pltpu.SemaphoreType.DMA,
    ]

    @functools.partial(
        pl.kernel,
        mesh=mesh,
        out_type=[
            jax.ShapeDtypeStruct((2, np_, D), jnp.float32),
            jax.ShapeDtypeStruct((2 * 4 * np_,), jnp.float32),
        ],
        scratch_types=buf_types + buf_types + [
            pltpu.VMEM((CHUNK,), jnp.float32),
            pltpu.VMEM((rows_per_tile,), jnp.float32),
            pltpu.VMEM_SHARED((np_, D), jnp.float32),
            pltpu.VMEM_SHARED((np_,), jnp.float32),
            pltpu.VMEM_SHARED((np_,), jnp.float32),
            pltpu.VMEM_SHARED((np_,), jnp.float32),
            pltpu.VMEM_SHARED((np_,), jnp.float32),
        ],
    )
    def scatter_k(msg_hbm, pw_hbm, diff_hbm, dst_hbm,
                  magg_hbm, pagg_hbm, *scratch):
        bufs = (scratch[:9], scratch[9:18])
        ones_v, z1_v, acc_msg, acc_x, acc_y, acc_z, acc_n = scratch[18:]
        z_v = bufs[0][1]
        accs = [acc_x, acc_y, acc_z, acc_n]
        cid = lax.axis_index("c")
        sid = lax.axis_index("s")
        wid = sid * 2 + cid
        nloc = base_p + jnp.where(wid < extra_p, 1, 0)
        pstart = wid * base_p + jnp.minimum(wid, extra_p)

        zero16 = jnp.zeros((LANES,), jnp.float32)
        one16 = jnp.ones((LANES,), jnp.float32)

        def fill_body(r, carry):
            for u in range(D // LANES):
                z_v[r, pl.ds(u * LANES, LANES)] = zero16
            return carry

        lax.fori_loop(0, CHUNK, fill_body, 0)

        def fill1_body(r, carry):
            z1_v[pl.ds(r * LANES, LANES)] = zero16
            return carry

        lax.fori_loop(0, rows_per_tile // LANES, fill1_body, 0)
        for u in range(CHUNK // LANES):
            ones_v[pl.ds(u * LANES, LANES)] = one16

        rbase = sid * rows_per_tile
        for k in range(rows_per_tile // CHUNK):
            pltpu.sync_copy(z_v, acc_msg.at[pl.ds(rbase + k * CHUNK, CHUNK)])
        for acc in accs:
            pltpu.sync_copy(z1_v, acc.at[pl.ds(rbase, rows_per_tile)])
        plsc.subcore_barrier()

        def drain_adds(b):
            idx_d, rows_v, pw_v, diff_v, vx_v, vy_v, vz_v, sem_r, sem_a = bufs[b]
            pltpu.make_async_copy(rows_v, acc_msg.at[pl.ds(0, CHUNK)],
                                  sem_a).wait()
            for v, acc in zip([vx_v, vy_v, vz_v, ones_v], accs):
                pltpu.make_async_copy(v, acc.at[pl.ds(0, CHUNK)], sem_a).wait()

        def pair_body(g, carry):
            eb0 = (pstart + g) * 2 * CHUNK
            read_cps = []
            for b in (0, 1):
                idx_d, rows_v, pw_v, diff_v, vx_v, vy_v, vz_v, sem_r, sem_a = bufs[b]
                eb = eb0 + b * CHUNK

                @pl.when(g > 0)
                def _():
                    drain_adds(b)

                read_cps.append([
                    pltpu.async_copy(dst_hbm.at[pl.ds(eb, CHUNK)], idx_d, sem_r),
                    pltpu.async_copy(msg_hbm.at[pl.ds(eb, CHUNK)], rows_v, sem_r),
                    pltpu.async_copy(pw_hbm.at[:, pl.ds(eb, CHUNK)], pw_v, sem_r),
                    pltpu.async_copy(diff_hbm.at[:, pl.ds(eb, CHUNK)], diff_v, sem_r),
                ])

            for b in (0, 1):
                idx_d, rows_v, pw_v, diff_v, vx_v, vy_v, vz_v, sem_r, sem_a = bufs[b]
                for cp in read_cps[b]:
                    cp.wait()
                for u in range(CHUNK // LANES):
                    sl = pl.ds(u * LANES, LANES)
                    pwg = pw_v[0, sl]
                    vx_v[sl] = diff_v[0, sl] * pwg
                    vy_v[sl] = diff_v[1, sl] * pwg
                    vz_v[sl] = diff_v[2, sl] * pwg
                pltpu.async_copy(rows_v, acc_msg.at[idx_d], sem_a, add=True)
                pltpu.async_copy(vx_v, acc_x.at[idx_d], sem_a, add=True)
                pltpu.async_copy(vy_v, acc_y.at[idx_d], sem_a, add=True)
                pltpu.async_copy(vz_v, acc_z.at[idx_d], sem_a, add=True)
                pltpu.async_copy(ones_v, acc_n.at[idx_d], sem_a, add=True)
            return carry

        lax.fori_loop(0, nloc, pair_body, 0)
        drain_adds(0)
        drain_adds(1)
        plsc.subcore_barrier()
        pltpu.sync_copy(acc_msg.at[pl.ds(rbase, rows_per_tile)],
                        magg_hbm.at[cid, pl.ds(rbase, rows_per_tile)])
        for cc, acc in enumerate(accs):
            pltpu.sync_copy(acc.at[pl.ds(rbase, rows_per_tile)],
                            pagg_hbm.at[pl.ds((cid * 4 + cc) * np_ + rbase,
                                              rows_per_tile)])

    return scatter_k


def _node_body(h_ref, pos_ref, magg_ref, pagg_ref,
               wu1h_ref, wu1m_ref, bu1_ref, gu1_ref, cu1_ref,
               wu2_ref, bu2_ref, gu2_ref, cu2_ref,
               w1a_ref, w1b_ref, b1_ref,
               hn_ref, posn_ref, a_ref, b_ref):
    h = h_ref[...]
    magg = magg_ref[0] + magg_ref[1]
    pagg = pagg_ref[0] + pagg_ref[1]
    lanes = lax.broadcasted_iota(jnp.int32, (1, 4), 1)
    cnt = jnp.sum(pagg * (lanes == 3).astype(jnp.float32), axis=-1, keepdims=True)
    cnt = jnp.maximum(cnt, 1.0)
    posd = pagg * (lanes < 3).astype(jnp.float32) / cnt
    posn_ref[...] = pos_ref[...] + posd

    u = (jnp.dot(h, wu1h_ref[...], preferred_element_type=jnp.float32)
         + jnp.dot(magg, wu1m_ref[...], preferred_element_type=jnp.float32)
         + bu1_ref[...])
    u = jnp.maximum(_ln(u, gu1_ref[...], cu1_ref[...]), 0.0)
    u = jnp.dot(u, wu2_ref[...], preferred_element_type=jnp.float32) + bu2_ref[...]
    u = jnp.maximum(_ln(u, gu2_ref[...], cu2_ref[...]), 0.0)
    hn = h + u
    hn_ref[...] = hn
    a_ref[...] = jnp.dot(hn, w1a_ref[...], preferred_element_type=jnp.float32) + b1_ref[...]
    b_ref[...] = jnp.dot(hn, w1b_ref[...], preferred_element_type=jnp.float32)


def _tc_node(h, pos4, magg_p, pagg_p, wts, n, bn):
    grid = (n // bn,)
    full = lambda i: (0, 0)
    return pl.pallas_call(
        _node_body,
        grid=grid,
        in_specs=[
            pl.BlockSpec((bn, D), lambda i: (i, 0)),
            pl.BlockSpec((bn, 4), lambda i: (i, 0)),
            pl.BlockSpec((2, bn, D), lambda i: (0, i, 0)),
            pl.BlockSpec((2, bn, 4), lambda i: (0, i, 0)),
        ] + [pl.BlockSpec(w.shape, full) for w in wts],
        out_specs=[
            pl.BlockSpec((bn, D), lambda i: (i, 0)),
            pl.BlockSpec((bn, 4), lambda i: (i, 0)),
            pl.BlockSpec((bn, D), lambda i: (i, 0)),
            pl.BlockSpec((bn, D), lambda i: (i, 0)),
        ],
        out_shape=[
            jax.ShapeDtypeStruct((n, D), jnp.float32),
            jax.ShapeDtypeStruct((n, 4), jnp.float32),
            jax.ShapeDtypeStruct((n, D), jnp.float32),
            jax.ShapeDtypeStruct((n, D), jnp.float32),
        ],
    )(h, pos4, magg_p, pagg_p, *wts)


def kernel(atoms, pos, edge_index, params):
    n = atoms.shape[0]
    e = edge_index.shape[1]
    layers = params["layers"]

    emb = params["embedding"]
    emb_pad = jnp.zeros((D, D), jnp.float32).at[: emb.shape[0]].set(emb)

    def msg1_parts(lp):
        w1 = lp["msg_l1"]["W"]
        return (w1[:D], w1[D:2 * D], w1[2 * D:2 * D + 1],
                lp["msg_l1"]["b"].reshape(1, D))

    src = edge_index[0].astype(jnp.int32)
    dst = edge_index[1].astype(jnp.int32)
    posf = pos.astype(jnp.float32)
    pos4 = jnp.pad(posf, ((0, 0), (0, 4 - pos.shape[1])))

    bn = 2000
    be = 2560
    w1a0, w1b0, _, b10 = msg1_parts(layers[0])
    h, a_cur, b_cur = _tc_init(atoms.astype(jnp.int32), emb_pad,
                               w1a0, w1b0, b10, n, bn)

    np_ = ((n + NW * 8 - 1) // (NW * 8)) * NW * 8
    gather_k = _make_sc_gather(n, e)
    scatter_k = _make_sc_scatter(np_, e)

    for li, lp in enumerate(layers):
        px, py, pz = pos4[:, 0], pos4[:, 1], pos4[:, 2]
        msg1p, diff3 = gather_k(a_cur, b_cur, px, py, pz, src, dst)

        wd = msg1_parts(lp)[2]
        ewts = [
            wd,
            lp["msg_ln1"]["g"].reshape(1, D), lp["msg_ln1"]["b"].reshape(1, D),
            lp["msg_l2"]["W"], lp["msg_l2"]["b"].reshape(1, D),
            lp["msg_ln2"]["g"].reshape(1, D), lp["msg_ln2"]["b"].reshape(1, D),
            lp["pos_l1"]["W"], lp["pos_l1"]["b"].reshape(1, D),
            lp["pos_ln1"]["g"].reshape(1, D), lp["pos_ln1"]["b"].reshape(1, D),
            lp["pos_l2"]["W"].reshape(1, D), lp["pos_l2"]["b"].reshape(1, 1),
        ]
        msg, pw = _tc_edge(msg1p, diff3, ewts, e, be)

        magg_p, pagg_flat = scatter_k(msg, pw, diff3, dst)
        pagg_p = pagg_flat.reshape(2, 4, np_).transpose(0, 2, 1)

        nxt = layers[li + 1] if li + 1 < len(layers) else layers[0]
        w1a_n, w1b_n, _, b1_n = msg1_parts(nxt)
        wu1 = lp["upd_l1"]["W"]
        nwts = [
            wu1[:D], wu1[D:], lp["upd_l1"]["b"].reshape(1, D),
            lp["upd_ln1"]["g"].reshape(1, D), lp["upd_ln1"]["b"].reshape(1, D),
            lp["upd_l2"]["W"], lp["upd_l2"]["b"].reshape(1, D),
            lp["upd_ln2"]["g"].reshape(1, D), lp["upd_ln2"]["b"].reshape(1, D),
            w1a_n, w1b_n, b1_n,
        ]
        h, pos4, a_cur, b_cur = _tc_node(h, pos4, magg_p, pagg_p, nwts, n, bn)

    return pos4[:, :3]


# speedup vs baseline: 4.9471x; 1.2266x over previous
import functools

import jax
import jax.numpy as jnp
from jax import lax
from jax.experimental import pallas as pl
from jax.experimental.pallas import tpu as pltpu
from jax.experimental.pallas import tpu_sc as plsc

D = 128
LANES = 16
CHUNK = 128
NW = 32


def _ln(x, g, b):
    m = jnp.mean(x, axis=-1, keepdims=True)
    v = jnp.mean((x - m) ** 2, axis=-1, keepdims=True)
    return (x - m) / jnp.sqrt(v + 1e-5) * g + b


def _init_body(atoms_ref, emb_ref, w1a_ref, w1b_ref, b1_ref,
               h_ref, a_ref, b_ref):
    at = atoms_ref[...]
    lanes = lax.broadcasted_iota(jnp.int32, (1, D), 1)
    oh = (at == lanes).astype(jnp.float32)
    h = jnp.dot(oh, emb_ref[...], preferred_element_type=jnp.float32)
    h_ref[...] = h
    a_ref[...] = jnp.dot(h, w1a_ref[...], preferred_element_type=jnp.float32) + b1_ref[...]
    b_ref[...] = jnp.dot(h, w1b_ref[...], preferred_element_type=jnp.float32)


def _tc_init(atoms, emb_pad, w1a, w1b, b1, n, bn):
    grid = (n // bn,)
    return pl.pallas_call(
        _init_body,
        grid=grid,
        in_specs=[
            pl.BlockSpec((bn, 1), lambda i: (i, 0)),
            pl.BlockSpec((D, D), lambda i: (0, 0)),
            pl.BlockSpec((D, D), lambda i: (0, 0)),
            pl.BlockSpec((D, D), lambda i: (0, 0)),
            pl.BlockSpec((1, D), lambda i: (0, 0)),
        ],
        out_specs=[
            pl.BlockSpec((bn, D), lambda i: (i, 0)),
            pl.BlockSpec((bn, D), lambda i: (i, 0)),
            pl.BlockSpec((bn, D), lambda i: (i, 0)),
        ],
        out_shape=[
            jax.ShapeDtypeStruct((n, D), jnp.float32),
            jax.ShapeDtypeStruct((n, D), jnp.float32),
            jax.ShapeDtypeStruct((n, D), jnp.float32),
        ],
    )(atoms, emb_pad, w1a, w1b, b1)


def _make_sc_gather(n, e):
    npairs = e // (2 * CHUNK)
    base_p = npairs // NW
    extra_p = npairs % NW
    mesh = plsc.VectorSubcoreMesh(core_axis_name="c", subcore_axis_name="s")

    buf_types = [
        pltpu.VMEM((CHUNK, D), jnp.float32),
        pltpu.VMEM((CHUNK, D), jnp.float32),
        pltpu.VMEM((3, CHUNK), jnp.float32),
        pltpu.SemaphoreType.DMA,
        pltpu.SemaphoreType.DMA,
    ]
    idx_types = [
        pltpu.VMEM((2, CHUNK), jnp.int32),
        pltpu.VMEM((2, CHUNK), jnp.int32),
        pltpu.SemaphoreType.DMA,
    ]

    @functools.partial(
        pl.kernel,
        mesh=mesh,
        out_type=[
            jax.ShapeDtypeStruct((e, D), jnp.float32),
            jax.ShapeDtypeStruct((3, e), jnp.float32),
        ],
        scratch_types=buf_types + buf_types + idx_types + idx_types + [
            pltpu.VMEM((2, CHUNK), jnp.float32),
            pltpu.VMEM((2, CHUNK), jnp.float32),
            pltpu.VMEM((2, CHUNK), jnp.float32),
            pltpu.VMEM((2, CHUNK), jnp.float32),
            pltpu.VMEM((2, CHUNK), jnp.float32),
            pltpu.VMEM((2, CHUNK), jnp.float32),
            pltpu.SemaphoreType.DMA,
        ],
    )
    def gather_k(a_hbm, b_hbm, px_hbm, py_hbm, pz_hbm, src2_hbm, dst2_hbm,
                 msg1p_hbm, diff_hbm, *scratch):
        bufs = (scratch[:5], scratch[5:10])
        ibufs = (scratch[10:13], scratch[13:16])
        gxi, gyi, gzi, gxj, gyj, gzj, sem_p = scratch[16:]
        cid = lax.axis_index("c")
        sid = lax.axis_index("s")
        wid = sid * 2 + cid
        nloc = base_p + jnp.where(wid < extra_p, 1, 0)
        pstart = wid * base_p + jnp.minimum(wid, extra_p)

        def issue_idx(p, pr):
            idx_d, idx_s, sem_i = ibufs[p]
            return (pltpu.async_copy(dst2_hbm.at[pl.ds(pr * 2, 2)], idx_d, sem_i),
                    pltpu.async_copy(src2_hbm.at[pl.ds(pr * 2, 2)], idx_s, sem_i))

        def drain_writes(b):
            rows_a, rows_b, dxyz, sem_g, sem_w = bufs[b]
            pltpu.make_async_copy(rows_a, msg1p_hbm.at[pl.ds(0, CHUNK)],
                                  sem_w).wait()
            pltpu.make_async_copy(dxyz, diff_hbm.at[:, pl.ds(0, CHUNK)],
                                  sem_w).wait()

        i0, i1 = issue_idx(0, pstart)
        i0.wait()
        i1.wait()

        def pair_body(g, carry):
            eb0 = (pstart + g) * 2 * CHUNK
            idx_d, idx_s, sem_i = ibufs[0]

            row_cps = []
            for b in (0, 1):
                rows_a, rows_b, dxyz, sem_g, sem_w = bufs[b]

                @pl.when(g > 0)
                def _():
                    drain_writes(b)

                row_cps.append(
                    (pltpu.async_copy(a_hbm.at[idx_d.at[b]], rows_a, sem_g),
                     pltpu.async_copy(b_hbm.at[idx_s.at[b]], rows_b, sem_g)))

            pos_cps = []
            for b in (0, 1):
                pos_cps += [
                    pltpu.async_copy(px_hbm.at[idx_d.at[b]], gxi.at[b], sem_p),
                    pltpu.async_copy(py_hbm.at[idx_d.at[b]], gyi.at[b], sem_p),
                    pltpu.async_copy(pz_hbm.at[idx_d.at[b]], gzi.at[b], sem_p),
                    pltpu.async_copy(px_hbm.at[idx_s.at[b]], gxj.at[b], sem_p),
                    pltpu.async_copy(py_hbm.at[idx_s.at[b]], gyj.at[b], sem_p),
                    pltpu.async_copy(pz_hbm.at[idx_s.at[b]], gzj.at[b], sem_p),
                ]

            nidx_d, nidx_s, nsem_i = ibufs[1]

            @pl.when(g + 1 < nloc)
            def _():
                pr = pstart + g + 1
                pltpu.async_copy(dst2_hbm.at[pl.ds(pr * 2, 2)], nidx_d, nsem_i)
                pltpu.async_copy(src2_hbm.at[pl.ds(pr * 2, 2)], nidx_s, nsem_i)

            for cp in pos_cps:
                cp.wait()
            for b in (0, 1):
                rows_a, rows_b, dxyz, sem_g, sem_w = bufs[b]
                eb = eb0 + b * CHUNK
                for u in range(CHUNK // LANES):
                    sl = pl.ds(u * LANES, LANES)
                    dxyz[0, sl] = gxi[b, sl] - gxj[b, sl]
                    dxyz[1, sl] = gyi[b, sl] - gyj[b, sl]
                    dxyz[2, sl] = gzi[b, sl] - gzj[b, sl]
                pltpu.async_copy(dxyz, diff_hbm.at[:, pl.ds(eb, CHUNK)], sem_w)
                row_cps[b][0].wait()
                row_cps[b][1].wait()

                def row_body(r, carry2):
                    for u in range(D // LANES):
                        sl = pl.ds(u * LANES, LANES)
                        rows_a[r, sl] = rows_a[r, sl] + rows_b[r, sl]
                    return carry2

                lax.fori_loop(0, CHUNK, row_body, 0)
                pltpu.async_copy(rows_a, msg1p_hbm.at[pl.ds(eb, CHUNK)], sem_w)

            @pl.when(g + 1 < nloc)
            def _():
                pltpu.make_async_copy(dst2_hbm.at[pl.ds(0, 2)], nidx_d,
                                      nsem_i).wait()
                pltpu.make_async_copy(src2_hbm.at[pl.ds(0, 2)], nidx_s,
                                      nsem_i).wait()
                for r in range(2):
                    for u in range(CHUNK // LANES):
                        sl = pl.ds(u * LANES, LANES)
                        idx_d[r, sl] = nidx_d[r, sl]
                        idx_s[r, sl] = nidx_s[r, sl]
            return carry

        lax.fori_loop(0, nloc, pair_body, 0)
        drain_writes(0)
        drain_writes(1)

    return gather_k


def _edge_body(msg1p_ref, diff_ref, wd_ref,
               g1_ref, c1_ref, w2_ref, b2_ref, g2_ref, c2_ref,
               wp1_ref, bp1_ref, gp_ref, cp_ref, wp2_ref, bp2_ref,
               msg_ref, pw_ref):
    dT = jnp.transpose(diff_ref[...])
    dist = jnp.sqrt(jnp.sum(dT * dT, axis=-1, keepdims=True))
    x = msg1p_ref[...].astype(jnp.float32) + dist * wd_ref[...]
    x = jnp.maximum(_ln(x, g1_ref[...], c1_ref[...]), 0.0)
    x = jnp.dot(x, w2_ref[...], preferred_element_type=jnp.float32) + b2_ref[...]
    msg = jnp.maximum(_ln(x, g2_ref[...], c2_ref[...]), 0.0)
    msg_ref[...] = msg
    p = jnp.dot(msg, wp1_ref[...], preferred_element_type=jnp.float32) + bp1_ref[...]
    p = jnp.maximum(_ln(p, gp_ref[...], cp_ref[...]), 0.0)
    pw = jnp.sum(p * wp2_ref[...], axis=-1) + bp2_ref[0, 0]
    pw_ref[...] = pw.reshape(1, -1)


def _tc_edge(msg1p, diff, wts, e, be):
    grid = (e // be,)
    full = lambda i: (0, 0)
    return pl.pallas_call(
        _edge_body,
        grid=grid,
        in_specs=[
            pl.BlockSpec((be, D), lambda i: (i, 0)),
            pl.BlockSpec((3, be), lambda i: (0, i)),
        ] + [pl.BlockSpec(w.shape, full) for w in wts],
        out_specs=[
            pl.BlockSpec((be, D), lambda i: (i, 0)),
            pl.BlockSpec((1, be), lambda i: (0, i)),
        ],
        out_shape=[
            jax.ShapeDtypeStruct((e, D), jnp.float32),
            jax.ShapeDtypeStruct((1, e), jnp.float32),
        ],
    )(msg1p, diff, *wts)


def _make_sc_scatter(np_, e):
    npairs = e // (2 * CHUNK)
    base_p = npairs // NW
    extra_p = npairs % NW
    rows_per_tile = np_ // 16
    mesh = plsc.VectorSubcoreMesh(core_axis_name="c", subcore_axis_name="s")

    buf_types = [
        pltpu.VMEM((CHUNK,), jnp.int32),
        pltpu.VMEM((CHUNK, D), jnp.float32),
        pltpu.VMEM((1, CHUNK), jnp.float32),
        pltpu.VMEM((3, CHUNK), jnp.float32),
        pltpu.VMEM((CHUNK,), jnp.float32),
        pltpu.VMEM((CHUNK,), jnp.float32),
        pltpu.VMEM((CHUNK,), jnp.float32),
        pltpu.SemaphoreType.DMA,
        pltpu.SemaphoreType.DMA,
    ]

    @functools.partial(
        pl.kernel,
        mesh=mesh,
        out_type=[
            jax.ShapeDtypeStruct((2, np_, D), jnp.float32),
            jax.ShapeDtypeStruct((2 * 4 * np_,), jnp.float32),
        ],
        scratch_types=buf_types + buf_types + [
            pltpu.VMEM((CHUNK,), jnp.float32),
            pltpu.VMEM((rows_per_tile,), jnp.float32),
            pltpu.VMEM_SHARED((np_, D), jnp.float32),
            pltpu.VMEM_SHARED((np_,), jnp.float32),
            pltpu.VMEM_SHARED((np_,), jnp.float32),
            pltpu.VMEM_SHARED((np_,), jnp.float32),
            pltpu.VMEM_SHARED((np_,), jnp.float32),
        ],
    )
    def scatter_k(msg_hbm, pw_hbm, diff_hbm, dst_hbm,
                  magg_hbm, pagg_hbm, *scratch):
        bufs = (scratch[:9], scratch[9:18])
        ones_v, z1_v, acc_msg, acc_x, acc_y, acc_z, acc_n = scratch[18:]
        z_v = bufs[0][1]
        accs = [acc_x, acc_y, acc_z, acc_n]
        cid = lax.axis_index("c")
        sid = lax.axis_index("s")
        wid = sid * 2 + cid
        nloc = base_p + jnp.where(wid < extra_p, 1, 0)
        pstart = wid * base_p + jnp.minimum(wid, extra_p)

        zero16 = jnp.zeros((LANES,), jnp.float32)
        one16 = jnp.ones((LANES,), jnp.float32)

        def fill_body(r, carry):
            for u in range(D // LANES):
                z_v[r, pl.ds(u * LANES, LANES)] = zero16
            return carry

        lax.fori_loop(0, CHUNK, fill_body, 0)

        def fill1_body(r, carry):
            z1_v[pl.ds(r * LANES, LANES)] = zero16
            return carry

        lax.fori_loop(0, rows_per_tile // LANES, fill1_body, 0)
        for u in range(CHUNK // LANES):
            ones_v[pl.ds(u * LANES, LANES)] = one16

        rbase = sid * rows_per_tile
        for k in range(rows_per_tile // CHUNK):
            pltpu.sync_copy(z_v, acc_msg.at[pl.ds(rbase + k * CHUNK, CHUNK)])
        for acc in accs:
            pltpu.sync_copy(z1_v, acc.at[pl.ds(rbase, rows_per_tile)])
        plsc.subcore_barrier()

        def drain_adds(b):
            idx_d, rows_v, pw_v, diff_v, vx_v, vy_v, vz_v, sem_r, sem_a = bufs[b]
            pltpu.make_async_copy(rows_v, acc_msg.at[pl.ds(0, CHUNK)],
                                  sem_a).wait()
            for v, acc in zip([vx_v, vy_v, vz_v, ones_v], accs):
                pltpu.make_async_copy(v, acc.at[pl.ds(0, CHUNK)], sem_a).wait()

        def pair_body(g, carry):
            eb0 = (pstart + g) * 2 * CHUNK
            read_cps = []
            for b in (0, 1):
                idx_d, rows_v, pw_v, diff_v, vx_v, vy_v, vz_v, sem_r, sem_a = bufs[b]
                eb = eb0 + b * CHUNK

                @pl.when(g > 0)
                def _():
                    drain_adds(b)

                read_cps.append([
                    pltpu.async_copy(dst_hbm.at[pl.ds(eb, CHUNK)], idx_d, sem_r),
                    pltpu.async_copy(msg_hbm.at[pl.ds(eb, CHUNK)], rows_v, sem_r),
                    pltpu.async_copy(pw_hbm.at[:, pl.ds(eb, CHUNK)], pw_v, sem_r),
                    pltpu.async_copy(diff_hbm.at[:, pl.ds(eb, CHUNK)], diff_v, sem_r),
                ])

            for b in (0, 1):
                idx_d, rows_v, pw_v, diff_v, vx_v, vy_v, vz_v, sem_r, sem_a = bufs[b]
                eb = eb0 + b * CHUNK
                for cp in read_cps[b]:
                    cp.wait()
                for u in range(CHUNK // LANES):
                    sl = pl.ds(u * LANES, LANES)
                    pwg = pw_v[0, sl]
                    vx_v[sl] = diff_v[0, sl] * pwg
                    vy_v[sl] = diff_v[1, sl] * pwg
                    vz_v[sl] = diff_v[2, sl] * pwg
                pltpu.async_copy(rows_v, acc_msg.at[idx_d], sem_a, add=True)
                pltpu.async_copy(vx_v, acc_x.at[idx_d], sem_a, add=True)
                pltpu.async_copy(vy_v, acc_y.at[idx_d], sem_a, add=True)
                pltpu.async_copy(vz_v, acc_z.at[idx_d], sem_a, add=True)
                pltpu.async_copy(ones_v, acc_n.at[idx_d], sem_a, add=True)
            return carry

        lax.fori_loop(0, nloc, pair_body, 0)
        drain_adds(0)
        drain_adds(1)
        plsc.subcore_barrier()
        pltpu.sync_copy(acc_msg.at[pl.ds(rbase, rows_per_tile)],
                        magg_hbm.at[cid, pl.ds(rbase, rows_per_tile)])
        for cc, acc in enumerate(accs):
            pltpu.sync_copy(acc.at[pl.ds(rbase, rows_per_tile)],
                            pagg_hbm.at[pl.ds((cid * 4 + cc) * np_ + rbase,
                                              rows_per_tile)])

    return scatter_k


def _node_body(h_ref, pos_ref, magg_ref, pagg_ref, maggb_ref, paggb_ref,
               wu1h_ref, wu1m_ref, bu1_ref, gu1_ref, cu1_ref,
               wu2_ref, bu2_ref, gu2_ref, cu2_ref,
               w1a_ref, w1b_ref, b1_ref,
               hn_ref, posn_ref, a_ref, b_ref):
    h = h_ref[...]
    magg = magg_ref[0] + magg_ref[1] + maggb_ref[0] + maggb_ref[1]
    pagg = pagg_ref[0] + pagg_ref[1] + paggb_ref[0] + paggb_ref[1]
    lanes = lax.broadcasted_iota(jnp.int32, (1, 4), 1)
    cnt = jnp.sum(pagg * (lanes == 3).astype(jnp.float32), axis=-1, keepdims=True)
    cnt = jnp.maximum(cnt, 1.0)
    posd = pagg * (lanes < 3).astype(jnp.float32) / cnt
    posn_ref[...] = pos_ref[...] + posd

    u = (jnp.dot(h, wu1h_ref[...], preferred_element_type=jnp.float32)
         + jnp.dot(magg, wu1m_ref[...], preferred_element_type=jnp.float32)
         + bu1_ref[...])
    u = jnp.maximum(_ln(u, gu1_ref[...], cu1_ref[...]), 0.0)
    u = jnp.dot(u, wu2_ref[...], preferred_element_type=jnp.float32) + bu2_ref[...]
    u = jnp.maximum(_ln(u, gu2_ref[...], cu2_ref[...]), 0.0)
    hn = h + u
    hn_ref[...] = hn
    a_ref[...] = jnp.dot(hn, w1a_ref[...], preferred_element_type=jnp.float32) + b1_ref[...]
    b_ref[...] = jnp.dot(hn, w1b_ref[...], preferred_element_type=jnp.float32)


def _tc_node(h, pos4, magg_p, pagg_p, magg_pb, pagg_pb, wts, n, bn):
    grid = (n // bn,)
    full = lambda i: (0, 0)
    return pl.pallas_call(
        _node_body,
        grid=grid,
        in_specs=[
            pl.BlockSpec((bn, D), lambda i: (i, 0)),
            pl.BlockSpec((bn, 4), lambda i: (i, 0)),
            pl.BlockSpec((2, bn, D), lambda i: (0, i, 0)),
            pl.BlockSpec((2, bn, 4), lambda i: (0, i, 0)),
            pl.BlockSpec((2, bn, D), lambda i: (0, i, 0)),
            pl.BlockSpec((2, bn, 4), lambda i: (0, i, 0)),
        ] + [pl.BlockSpec(w.shape, full) for w in wts],
        out_specs=[
            pl.BlockSpec((bn, D), lambda i: (i, 0)),
            pl.BlockSpec((bn, 4), lambda i: (i, 0)),
            pl.BlockSpec((bn, D), lambda i: (i, 0)),
            pl.BlockSpec((bn, D), lambda i: (i, 0)),
        ],
        out_shape=[
            jax.ShapeDtypeStruct((n, D), jnp.float32),
            jax.ShapeDtypeStruct((n, 4), jnp.float32),
            jax.ShapeDtypeStruct((n, D), jnp.float32),
            jax.ShapeDtypeStruct((n, D), jnp.float32),
        ],
    )(h, pos4, magg_p, pagg_p, magg_pb, pagg_pb, *wts)


def _final_body(pos_ref, pagg_ref, paggb_ref, posn_ref):
    pagg = pagg_ref[0] + pagg_ref[1] + paggb_ref[0] + paggb_ref[1]
    lanes = lax.broadcasted_iota(jnp.int32, (1, 4), 1)
    cnt = jnp.sum(pagg * (lanes == 3).astype(jnp.float32), axis=-1, keepdims=True)
    cnt = jnp.maximum(cnt, 1.0)
    posd = pagg * (lanes < 3).astype(jnp.float32) / cnt
    posn_ref[...] = pos_ref[...] + posd


def _tc_final(pos4, pagg_p, pagg_pb, n, bn):
    grid = (n // bn,)
    return pl.pallas_call(
        _final_body,
        grid=grid,
        in_specs=[
            pl.BlockSpec((bn, 4), lambda i: (i, 0)),
            pl.BlockSpec((2, bn, 4), lambda i: (0, i, 0)),
            pl.BlockSpec((2, bn, 4), lambda i: (0, i, 0)),
        ],
        out_specs=pl.BlockSpec((bn, 4), lambda i: (i, 0)),
        out_shape=jax.ShapeDtypeStruct((n, 4), jnp.float32),
    )(pos4, pagg_p, pagg_pb)


def kernel(atoms, pos, edge_index, params):
    n = atoms.shape[0]
    e = edge_index.shape[1]
    layers = params["layers"]

    emb = params["embedding"]
    emb_pad = jnp.zeros((D, D), jnp.float32).at[: emb.shape[0]].set(emb)

    def msg1_parts(lp):
        w1 = lp["msg_l1"]["W"]
        return (w1[:D], w1[D:2 * D], w1[2 * D:2 * D + 1],
                lp["msg_l1"]["b"].reshape(1, D))

    src = edge_index[0].astype(jnp.int32)
    dst = edge_index[1].astype(jnp.int32)
    src2 = src.reshape(-1, CHUNK)
    dst2 = dst.reshape(-1, CHUNK)
    posf = pos.astype(jnp.float32)
    pos4 = jnp.pad(posf, ((0, 0), (0, 4 - pos.shape[1])))

    bn = 2000
    be = 3200
    w1a0, w1b0, _, b10 = msg1_parts(layers[0])
    h, a_cur, b_cur = _tc_init(atoms.astype(jnp.int32), emb_pad,
                               w1a0, w1b0, b10, n, bn)

    np_ = ((n + NW * 8 - 1) // (NW * 8)) * NW * 8
    eh = e // 2
    gather_k = _make_sc_gather(n, eh)
    scatter_k = _make_sc_scatter(np_, eh)
    nrows = e // CHUNK
    src2h = [src2[: nrows // 2], src2[nrows // 2:]]
    dst2h = [dst2[: nrows // 2], dst2[nrows // 2:]]
    dsth = [dst[:eh], dst[eh:]]

    for li, lp in enumerate(layers):
        px, py, pz = pos4[:, 0], pos4[:, 1], pos4[:, 2]

        wd = msg1_parts(lp)[2]
        ewts = [
            wd,
            lp["msg_ln1"]["g"].reshape(1, D), lp["msg_ln1"]["b"].reshape(1, D),
            lp["msg_l2"]["W"], lp["msg_l2"]["b"].reshape(1, D),
            lp["msg_ln2"]["g"].reshape(1, D), lp["msg_ln2"]["b"].reshape(1, D),
            lp["pos_l1"]["W"], lp["pos_l1"]["b"].reshape(1, D),
            lp["pos_ln1"]["g"].reshape(1, D), lp["pos_ln1"]["b"].reshape(1, D),
            lp["pos_l2"]["W"].reshape(1, D), lp["pos_l2"]["b"].reshape(1, 1),
        ]
        gh = [gather_k(a_cur, b_cur, px, py, pz, src2h[hf], dst2h[hf])
              for hf in (0, 1)]
        eh_out = [_tc_edge(gh[hf][0], gh[hf][1], ewts, eh, be) for hf in (0, 1)]
        sc = [scatter_k(eh_out[hf][0], eh_out[hf][1], gh[hf][1], dsth[hf])
              for hf in (0, 1)]
        pagg_ps = [s[1].reshape(2, 4, np_).transpose(0, 2, 1) for s in sc]

        if li + 1 < len(layers):
            w1a_n, w1b_n, _, b1_n = msg1_parts(layers[li + 1])
            wu1 = lp["upd_l1"]["W"]
            nwts = [
                wu1[:D], wu1[D:], lp["upd_l1"]["b"].reshape(1, D),
                lp["upd_ln1"]["g"].reshape(1, D), lp["upd_ln1"]["b"].reshape(1, D),
                lp["upd_l2"]["W"], lp["upd_l2"]["b"].reshape(1, D),
                lp["upd_ln2"]["g"].reshape(1, D), lp["upd_ln2"]["b"].reshape(1, D),
                w1a_n, w1b_n, b1_n,
            ]
            h, pos4, a_cur, b_cur = _tc_node(h, pos4, sc[0][0], pagg_ps[0],
                                             sc[1][0], pagg_ps[1], nwts, n, bn)
        else:
            pos4 = _tc_final(pos4, pagg_ps[0], pagg_ps[1], n, bn)

    return pos4[:, :3]


# speedup vs baseline: 5.0313x; 1.0170x over previous
import functools

import jax
import jax.numpy as jnp
from jax import lax
from jax.experimental import pallas as pl
from jax.experimental.pallas import tpu as pltpu
from jax.experimental.pallas import tpu_sc as plsc

D = 128
LANES = 16
CHUNK = 128
NW = 32


def _ln(x, g, b):
    m = jnp.mean(x, axis=-1, keepdims=True)
    v = jnp.mean((x - m) ** 2, axis=-1, keepdims=True)
    return (x - m) / jnp.sqrt(v + 1e-5) * g + b


def _init_body(atoms_ref, emb_ref, w1a_ref, w1b_ref, b1_ref,
               h_ref, a_ref, b_ref):
    at = atoms_ref[...]
    lanes = lax.broadcasted_iota(jnp.int32, (1, D), 1)
    oh = (at == lanes).astype(jnp.float32)
    h = jnp.dot(oh, emb_ref[...], preferred_element_type=jnp.float32)
    h_ref[...] = h
    a_ref[...] = jnp.dot(h, w1a_ref[...], preferred_element_type=jnp.float32) + b1_ref[...]
    b_ref[...] = jnp.dot(h, w1b_ref[...], preferred_element_type=jnp.float32)


def _tc_init(atoms, emb_pad, w1a, w1b, b1, n, bn):
    grid = (n // bn,)
    return pl.pallas_call(
        _init_body,
        grid=grid,
        in_specs=[
            pl.BlockSpec((bn, 1), lambda i: (i, 0)),
            pl.BlockSpec((D, D), lambda i: (0, 0)),
            pl.BlockSpec((D, D), lambda i: (0, 0)),
            pl.BlockSpec((D, D), lambda i: (0, 0)),
            pl.BlockSpec((1, D), lambda i: (0, 0)),
        ],
        out_specs=[
            pl.BlockSpec((bn, D), lambda i: (i, 0)),
            pl.BlockSpec((bn, D), lambda i: (i, 0)),
            pl.BlockSpec((bn, D), lambda i: (i, 0)),
        ],
        out_shape=[
            jax.ShapeDtypeStruct((n, D), jnp.float32),
            jax.ShapeDtypeStruct((n, D), jnp.float32),
            jax.ShapeDtypeStruct((n, D), jnp.float32),
        ],
    )(atoms, emb_pad, w1a, w1b, b1)


def _make_sc_gather(n, e):
    npairs = e // (2 * CHUNK)
    base_p = npairs // NW
    extra_p = npairs % NW
    mesh = plsc.VectorSubcoreMesh(core_axis_name="c", subcore_axis_name="s")

    buf_types = [
        pltpu.VMEM((CHUNK, D), jnp.float32),
        pltpu.VMEM((CHUNK, D), jnp.float32),
        pltpu.VMEM((3, CHUNK), jnp.float32),
        pltpu.SemaphoreType.DMA,
        pltpu.SemaphoreType.DMA,
    ]
    idx_types = [
        pltpu.VMEM((2, CHUNK), jnp.int32),
        pltpu.VMEM((2, CHUNK), jnp.int32),
        pltpu.SemaphoreType.DMA,
    ]

    @functools.partial(
        pl.kernel,
        mesh=mesh,
        out_type=[
            jax.ShapeDtypeStruct((e, D), jnp.float32),
            jax.ShapeDtypeStruct((3, e), jnp.float32),
        ],
        scratch_types=buf_types + buf_types + idx_types + idx_types + [
            pltpu.VMEM((2, CHUNK), jnp.float32),
            pltpu.VMEM((2, CHUNK), jnp.float32),
            pltpu.VMEM((2, CHUNK), jnp.float32),
            pltpu.VMEM((2, CHUNK), jnp.float32),
            pltpu.VMEM((2, CHUNK), jnp.float32),
            pltpu.VMEM((2, CHUNK), jnp.float32),
            pltpu.SemaphoreType.DMA,
        ],
    )
    def gather_k(a_hbm, b_hbm, px_hbm, py_hbm, pz_hbm, src2_hbm, dst2_hbm,
                 msg1p_hbm, diff_hbm, *scratch):
        bufs = (scratch[:5], scratch[5:10])
        ibufs = (scratch[10:13], scratch[13:16])
        gxi, gyi, gzi, gxj, gyj, gzj, sem_p = scratch[16:]
        cid = lax.axis_index("c")
        sid = lax.axis_index("s")
        wid = sid * 2 + cid
        nloc = base_p + jnp.where(wid < extra_p, 1, 0)
        pstart = wid * base_p + jnp.minimum(wid, extra_p)

        def issue_idx(p, pr):
            idx_d, idx_s, sem_i = ibufs[p]
            return (pltpu.async_copy(dst2_hbm.at[pl.ds(pr * 2, 2)], idx_d, sem_i),
                    pltpu.async_copy(src2_hbm.at[pl.ds(pr * 2, 2)], idx_s, sem_i))

        def drain_writes(b):
            rows_a, rows_b, dxyz, sem_g, sem_w = bufs[b]
            pltpu.make_async_copy(rows_a, msg1p_hbm.at[pl.ds(0, CHUNK)],
                                  sem_w).wait()
            pltpu.make_async_copy(dxyz, diff_hbm.at[:, pl.ds(0, CHUNK)],
                                  sem_w).wait()

        i0, i1 = issue_idx(0, pstart)
        i0.wait()
        i1.wait()

        def pair_body(g, carry):
            eb0 = (pstart + g) * 2 * CHUNK
            idx_d, idx_s, sem_i = ibufs[0]

            row_cps = []
            for b in (0, 1):
                rows_a, rows_b, dxyz, sem_g, sem_w = bufs[b]

                @pl.when(g > 0)
                def _():
                    drain_writes(b)

                row_cps.append(
                    (pltpu.async_copy(a_hbm.at[idx_d.at[b]], rows_a, sem_g),
                     pltpu.async_copy(b_hbm.at[idx_s.at[b]], rows_b, sem_g)))

            pos_cps = []
            for b in (0, 1):
                pos_cps += [
                    pltpu.async_copy(px_hbm.at[idx_d.at[b]], gxi.at[b], sem_p),
                    pltpu.async_copy(py_hbm.at[idx_d.at[b]], gyi.at[b], sem_p),
                    pltpu.async_copy(pz_hbm.at[idx_d.at[b]], gzi.at[b], sem_p),
                    pltpu.async_copy(px_hbm.at[idx_s.at[b]], gxj.at[b], sem_p),
                    pltpu.async_copy(py_hbm.at[idx_s.at[b]], gyj.at[b], sem_p),
                    pltpu.async_copy(pz_hbm.at[idx_s.at[b]], gzj.at[b], sem_p),
                ]

            nidx_d, nidx_s, nsem_i = ibufs[1]

            @pl.when(g + 1 < nloc)
            def _():
                pr = pstart + g + 1
                pltpu.async_copy(dst2_hbm.at[pl.ds(pr * 2, 2)], nidx_d, nsem_i)
                pltpu.async_copy(src2_hbm.at[pl.ds(pr * 2, 2)], nidx_s, nsem_i)

            for cp in pos_cps:
                cp.wait()
            for b in (0, 1):
                rows_a, rows_b, dxyz, sem_g, sem_w = bufs[b]
                eb = eb0 + b * CHUNK
                for u in range(CHUNK // LANES):
                    sl = pl.ds(u * LANES, LANES)
                    dxyz[0, sl] = gxi[b, sl] - gxj[b, sl]
                    dxyz[1, sl] = gyi[b, sl] - gyj[b, sl]
                    dxyz[2, sl] = gzi[b, sl] - gzj[b, sl]
                pltpu.async_copy(dxyz, diff_hbm.at[:, pl.ds(eb, CHUNK)], sem_w)
                row_cps[b][0].wait()
                row_cps[b][1].wait()

                def row_body(r, carry2):
                    for u in range(D // LANES):
                        sl = pl.ds(u * LANES, LANES)
                        rows_a[r, sl] = rows_a[r, sl] + rows_b[r, sl]
                    return carry2

                lax.fori_loop(0, CHUNK, row_body, 0)
                pltpu.async_copy(rows_a, msg1p_hbm.at[pl.ds(eb, CHUNK)], sem_w)

            @pl.when(g + 1 < nloc)
            def _():
                pltpu.make_async_copy(dst2_hbm.at[pl.ds(0, 2)], nidx_d,
                                      nsem_i).wait()
                pltpu.make_async_copy(src2_hbm.at[pl.ds(0, 2)], nidx_s,
                                      nsem_i).wait()
                for r in range(2):
                    for u in range(CHUNK // LANES):
                        sl = pl.ds(u * LANES, LANES)
                        idx_d[r, sl] = nidx_d[r, sl]
                        idx_s[r, sl] = nidx_s[r, sl]
            return carry

        lax.fori_loop(0, nloc, pair_body, 0)
        drain_writes(0)
        drain_writes(1)

    return gather_k


def _edge_body(msg1p_ref, diff_ref, wd_ref,
               g1_ref, c1_ref, w2_ref, b2_ref, g2_ref, c2_ref,
               wp1_ref, bp1_ref, gp_ref, cp_ref, wp2_ref, bp2_ref,
               msg_ref, pw_ref):
    dT = jnp.transpose(diff_ref[...])
    dist = jnp.sqrt(jnp.sum(dT * dT, axis=-1, keepdims=True))
    x = msg1p_ref[...].astype(jnp.float32) + dist * wd_ref[...]
    x = jnp.maximum(_ln(x, g1_ref[...], c1_ref[...]), 0.0)
    x = jnp.dot(x, w2_ref[...], preferred_element_type=jnp.float32) + b2_ref[...]
    msg = jnp.maximum(_ln(x, g2_ref[...], c2_ref[...]), 0.0)
    msg_ref[...] = msg
    p = jnp.dot(msg, wp1_ref[...], preferred_element_type=jnp.float32) + bp1_ref[...]
    p = jnp.maximum(_ln(p, gp_ref[...], cp_ref[...]), 0.0)
    pw = jnp.sum(p * wp2_ref[...], axis=-1) + bp2_ref[0, 0]
    pw_ref[...] = pw.reshape(1, -1)


def _tc_edge(msg1p, diff, wts, e, be):
    grid = (e // be,)
    full = lambda i: (0, 0)
    return pl.pallas_call(
        _edge_body,
        grid=grid,
        in_specs=[
            pl.BlockSpec((be, D), lambda i: (i, 0)),
            pl.BlockSpec((3, be), lambda i: (0, i)),
        ] + [pl.BlockSpec(w.shape, full) for w in wts],
        out_specs=[
            pl.BlockSpec((be, D), lambda i: (i, 0)),
            pl.BlockSpec((1, be), lambda i: (0, i)),
        ],
        out_shape=[
            jax.ShapeDtypeStruct((e, D), jnp.float32),
            jax.ShapeDtypeStruct((1, e), jnp.float32),
        ],
    )(msg1p, diff, *wts)


def _make_sc_scatter(np_, e):
    npairs = e // (2 * CHUNK)
    base_p = npairs // NW
    extra_p = npairs % NW
    rows_per_tile = np_ // 16
    mesh = plsc.VectorSubcoreMesh(core_axis_name="c", subcore_axis_name="s")

    buf_types = [
        pltpu.VMEM((CHUNK,), jnp.int32),
        pltpu.VMEM((CHUNK, D), jnp.float32),
        pltpu.VMEM((1, CHUNK), jnp.float32),
        pltpu.VMEM((3, CHUNK), jnp.float32),
        pltpu.VMEM((CHUNK,), jnp.float32),
        pltpu.VMEM((CHUNK,), jnp.float32),
        pltpu.VMEM((CHUNK,), jnp.float32),
        pltpu.SemaphoreType.DMA,
        pltpu.SemaphoreType.DMA,
    ]

    @functools.partial(
        pl.kernel,
        mesh=mesh,
        out_type=[
            jax.ShapeDtypeStruct((2, np_, D), jnp.float32),
            jax.ShapeDtypeStruct((2 * 4 * np_,), jnp.float32),
        ],
        scratch_types=buf_types + buf_types + [
            pltpu.VMEM((CHUNK,), jnp.float32),
            pltpu.VMEM((rows_per_tile,), jnp.float32),
            pltpu.VMEM_SHARED((np_, D), jnp.float32),
            pltpu.VMEM_SHARED((np_,), jnp.float32),
            pltpu.VMEM_SHARED((np_,), jnp.float32),
            pltpu.VMEM_SHARED((np_,), jnp.float32),
            pltpu.VMEM_SHARED((np_,), jnp.float32),
        ],
    )
    def scatter_k(msg_hbm, pw_hbm, diff_hbm, dst_hbm,
                  magg_hbm, pagg_hbm, *scratch):
        bufs = (scratch[:9], scratch[9:18])
        ones_v, z1_v, acc_msg, acc_x, acc_y, acc_z, acc_n = scratch[18:]
        z_v = bufs[0][1]
        accs = [acc_x, acc_y, acc_z, acc_n]
        cid = lax.axis_index("c")
        sid = lax.axis_index("s")
        wid = sid * 2 + cid
        nloc = base_p + jnp.where(wid < extra_p, 1, 0)
        pstart = wid * base_p + jnp.minimum(wid, extra_p)

        zero16 = jnp.zeros((LANES,), jnp.float32)
        one16 = jnp.ones((LANES,), jnp.float32)

        def fill_body(r, carry):
            for u in range(D // LANES):
                z_v[r, pl.ds(u * LANES, LANES)] = zero16
            return carry

        lax.fori_loop(0, CHUNK, fill_body, 0)

        def fill1_body(r, carry):
            z1_v[pl.ds(r * LANES, LANES)] = zero16
            return carry

        lax.fori_loop(0, rows_per_tile // LANES, fill1_body, 0)
        for u in range(CHUNK // LANES):
            ones_v[pl.ds(u * LANES, LANES)] = one16

        rbase = sid * rows_per_tile
        for k in range(rows_per_tile // CHUNK):
            pltpu.sync_copy(z_v, acc_msg.at[pl.ds(rbase + k * CHUNK, CHUNK)])
        for acc in accs:
            pltpu.sync_copy(z1_v, acc.at[pl.ds(rbase, rows_per_tile)])
        plsc.subcore_barrier()

        def drain_adds(b):
            idx_d, rows_v, pw_v, diff_v, vx_v, vy_v, vz_v, sem_r, sem_a = bufs[b]
            pltpu.make_async_copy(rows_v, acc_msg.at[pl.ds(0, CHUNK)],
                                  sem_a).wait()
            for v, acc in zip([vx_v, vy_v, vz_v, ones_v], accs):
                pltpu.make_async_copy(v, acc.at[pl.ds(0, CHUNK)], sem_a).wait()

        def pair_body(g, carry):
            eb0 = (pstart + g) * 2 * CHUNK
            read_cps = []
            for b in (0, 1):
                idx_d, rows_v, pw_v, diff_v, vx_v, vy_v, vz_v, sem_r, sem_a = bufs[b]
                eb = eb0 + b * CHUNK

                @pl.when(g > 0)
                def _():
                    drain_adds(b)

                read_cps.append([
                    pltpu.async_copy(dst_hbm.at[pl.ds(eb, CHUNK)], idx_d, sem_r),
                    pltpu.async_copy(msg_hbm.at[pl.ds(eb, CHUNK)], rows_v, sem_r),
                    pltpu.async_copy(pw_hbm.at[:, pl.ds(eb, CHUNK)], pw_v, sem_r),
                    pltpu.async_copy(diff_hbm.at[:, pl.ds(eb, CHUNK)], diff_v, sem_r),
                ])

            for b in (0, 1):
                idx_d, rows_v, pw_v, diff_v, vx_v, vy_v, vz_v, sem_r, sem_a = bufs[b]
                eb = eb0 + b * CHUNK
                for cp in read_cps[b]:
                    cp.wait()
                for u in range(CHUNK // LANES):
                    sl = pl.ds(u * LANES, LANES)
                    pwg = pw_v[0, sl]
                    vx_v[sl] = diff_v[0, sl] * pwg
                    vy_v[sl] = diff_v[1, sl] * pwg
                    vz_v[sl] = diff_v[2, sl] * pwg
                pltpu.async_copy(rows_v, acc_msg.at[idx_d], sem_a, add=True)
                pltpu.async_copy(vx_v, acc_x.at[idx_d], sem_a, add=True)
                pltpu.async_copy(vy_v, acc_y.at[idx_d], sem_a, add=True)
                pltpu.async_copy(vz_v, acc_z.at[idx_d], sem_a, add=True)
                pltpu.async_copy(ones_v, acc_n.at[idx_d], sem_a, add=True)
            return carry

        lax.fori_loop(0, nloc, pair_body, 0)
        drain_adds(0)
        drain_adds(1)
        plsc.subcore_barrier()
        pltpu.sync_copy(acc_msg.at[pl.ds(rbase, rows_per_tile)],
                        magg_hbm.at[cid, pl.ds(rbase, rows_per_tile)])
        for cc, acc in enumerate(accs):
            pltpu.sync_copy(acc.at[pl.ds(rbase, rows_per_tile)],
                            pagg_hbm.at[pl.ds((cid * 4 + cc) * np_ + rbase,
                                              rows_per_tile)])

    return scatter_k


def _edge_pw_body(msg1p_ref, diff_ref, wd_ref,
                  g1_ref, c1_ref, w2_ref, b2_ref, g2_ref, c2_ref,
                  wp1_ref, bp1_ref, gp_ref, cp_ref, wp2_ref, bp2_ref,
                  pw_ref):
    dT = jnp.transpose(diff_ref[...])
    dist = jnp.sqrt(jnp.sum(dT * dT, axis=-1, keepdims=True))
    x = msg1p_ref[...].astype(jnp.float32) + dist * wd_ref[...]
    x = jnp.maximum(_ln(x, g1_ref[...], c1_ref[...]), 0.0)
    x = jnp.dot(x, w2_ref[...], preferred_element_type=jnp.float32) + b2_ref[...]
    msg = jnp.maximum(_ln(x, g2_ref[...], c2_ref[...]), 0.0)
    p = jnp.dot(msg, wp1_ref[...], preferred_element_type=jnp.float32) + bp1_ref[...]
    p = jnp.maximum(_ln(p, gp_ref[...], cp_ref[...]), 0.0)
    pw = jnp.sum(p * wp2_ref[...], axis=-1) + bp2_ref[0, 0]
    pw_ref[...] = pw.reshape(1, -1)


def _tc_edge_pw(msg1p, diff, wts, e, be):
    grid = (e // be,)
    full = lambda i: (0, 0)
    return pl.pallas_call(
        _edge_pw_body,
        grid=grid,
        in_specs=[
            pl.BlockSpec((be, D), lambda i: (i, 0)),
            pl.BlockSpec((3, be), lambda i: (0, i)),
        ] + [pl.BlockSpec(w.shape, full) for w in wts],
        out_specs=pl.BlockSpec((1, be), lambda i: (0, i)),
        out_shape=jax.ShapeDtypeStruct((1, e), jnp.float32),
    )(msg1p, diff, *wts)


def _make_sc_scatter_pos(np_, e):
    npairs = e // (2 * CHUNK)
    base_p = npairs // NW
    extra_p = npairs % NW
    rows_per_tile = np_ // 16
    mesh = plsc.VectorSubcoreMesh(core_axis_name="c", subcore_axis_name="s")

    buf_types = [
        pltpu.VMEM((CHUNK,), jnp.int32),
        pltpu.VMEM((1, CHUNK), jnp.float32),
        pltpu.VMEM((3, CHUNK), jnp.float32),
        pltpu.VMEM((CHUNK,), jnp.float32),
        pltpu.VMEM((CHUNK,), jnp.float32),
        pltpu.VMEM((CHUNK,), jnp.float32),
        pltpu.SemaphoreType.DMA,
        pltpu.SemaphoreType.DMA,
    ]

    @functools.partial(
        pl.kernel,
        mesh=mesh,
        out_type=[
            jax.ShapeDtypeStruct((2 * 4 * np_,), jnp.float32),
        ],
        scratch_types=buf_types + buf_types + [
            pltpu.VMEM((CHUNK,), jnp.float32),
            pltpu.VMEM((rows_per_tile,), jnp.float32),
            pltpu.VMEM_SHARED((np_,), jnp.float32),
            pltpu.VMEM_SHARED((np_,), jnp.float32),
            pltpu.VMEM_SHARED((np_,), jnp.float32),
            pltpu.VMEM_SHARED((np_,), jnp.float32),
        ],
    )
    def scatter_pos_k(pw_hbm, diff_hbm, dst_hbm, pagg_hbm, *scratch):
        bufs = (scratch[:8], scratch[8:16])
        ones_v, z1_v, acc_x, acc_y, acc_z, acc_n = scratch[16:]
        accs = [acc_x, acc_y, acc_z, acc_n]
        cid = lax.axis_index("c")
        sid = lax.axis_index("s")
        wid = sid * 2 + cid
        nloc = base_p + jnp.where(wid < extra_p, 1, 0)
        pstart = wid * base_p + jnp.minimum(wid, extra_p)

        zero16 = jnp.zeros((LANES,), jnp.float32)
        one16 = jnp.ones((LANES,), jnp.float32)

        def fill1_body(r, carry):
            z1_v[pl.ds(r * LANES, LANES)] = zero16
            return carry

        lax.fori_loop(0, rows_per_tile // LANES, fill1_body, 0)
        for u in range(CHUNK // LANES):
            ones_v[pl.ds(u * LANES, LANES)] = one16

        rbase = sid * rows_per_tile
        for acc in accs:
            pltpu.sync_copy(z1_v, acc.at[pl.ds(rbase, rows_per_tile)])
        plsc.subcore_barrier()

        def drain_adds(b):
            idx_d, pw_v, diff_v, vx_v, vy_v, vz_v, sem_r, sem_a = bufs[b]
            for v, acc in zip([vx_v, vy_v, vz_v, ones_v], accs):
                pltpu.make_async_copy(v, acc.at[pl.ds(0, CHUNK)], sem_a).wait()

        def pair_body(g, carry):
            eb0 = (pstart + g) * 2 * CHUNK
            read_cps = []
            for b in (0, 1):
                idx_d, pw_v, diff_v, vx_v, vy_v, vz_v, sem_r, sem_a = bufs[b]
                eb = eb0 + b * CHUNK

                @pl.when(g > 0)
                def _():
                    drain_adds(b)

                read_cps.append([
                    pltpu.async_copy(dst_hbm.at[pl.ds(eb, CHUNK)], idx_d, sem_r),
                    pltpu.async_copy(pw_hbm.at[:, pl.ds(eb, CHUNK)], pw_v, sem_r),
                    pltpu.async_copy(diff_hbm.at[:, pl.ds(eb, CHUNK)], diff_v, sem_r),
                ])

            for b in (0, 1):
                idx_d, pw_v, diff_v, vx_v, vy_v, vz_v, sem_r, sem_a = bufs[b]
                for cp in read_cps[b]:
                    cp.wait()
                for u in range(CHUNK // LANES):
                    sl = pl.ds(u * LANES, LANES)
                    pwg = pw_v[0, sl]
                    vx_v[sl] = diff_v[0, sl] * pwg
                    vy_v[sl] = diff_v[1, sl] * pwg
                    vz_v[sl] = diff_v[2, sl] * pwg
                pltpu.async_copy(vx_v, acc_x.at[idx_d], sem_a, add=True)
                pltpu.async_copy(vy_v, acc_y.at[idx_d], sem_a, add=True)
                pltpu.async_copy(vz_v, acc_z.at[idx_d], sem_a, add=True)
                pltpu.async_copy(ones_v, acc_n.at[idx_d], sem_a, add=True)
            return carry

        lax.fori_loop(0, nloc, pair_body, 0)
        drain_adds(0)
        drain_adds(1)
        plsc.subcore_barrier()
        for cc, acc in enumerate(accs):
            pltpu.sync_copy(acc.at[pl.ds(rbase, rows_per_tile)],
                            pagg_hbm.at[pl.ds((cid * 4 + cc) * np_ + rbase,
                                              rows_per_tile)])

    return scatter_pos_k


def _node_body(h_ref, pos_ref, magg_ref, pagg_ref, maggb_ref, paggb_ref,
               wu1h_ref, wu1m_ref, bu1_ref, gu1_ref, cu1_ref,
               wu2_ref, bu2_ref, gu2_ref, cu2_ref,
               w1a_ref, w1b_ref, b1_ref,
               hn_ref, posn_ref, a_ref, b_ref):
    h = h_ref[...]
    magg = magg_ref[0] + magg_ref[1] + maggb_ref[0] + maggb_ref[1]
    pagg = pagg_ref[0] + pagg_ref[1] + paggb_ref[0] + paggb_ref[1]
    lanes = lax.broadcasted_iota(jnp.int32, (1, 4), 1)
    cnt = jnp.sum(pagg * (lanes == 3).astype(jnp.float32), axis=-1, keepdims=True)
    cnt = jnp.maximum(cnt, 1.0)
    posd = pagg * (lanes < 3).astype(jnp.float32) / cnt
    posn_ref[...] = pos_ref[...] + posd

    u = (jnp.dot(h, wu1h_ref[...], preferred_element_type=jnp.float32)
         + jnp.dot(magg, wu1m_ref[...], preferred_element_type=jnp.float32)
         + bu1_ref[...])
    u = jnp.maximum(_ln(u, gu1_ref[...], cu1_ref[...]), 0.0)
    u = jnp.dot(u, wu2_ref[...], preferred_element_type=jnp.float32) + bu2_ref[...]
    u = jnp.maximum(_ln(u, gu2_ref[...], cu2_ref[...]), 0.0)
    hn = h + u
    hn_ref[...] = hn
    a_ref[...] = jnp.dot(hn, w1a_ref[...], preferred_element_type=jnp.float32) + b1_ref[...]
    b_ref[...] = jnp.dot(hn, w1b_ref[...], preferred_element_type=jnp.float32)


def _tc_node(h, pos4, magg_p, pagg_p, magg_pb, pagg_pb, wts, n, bn):
    grid = (n // bn,)
    full = lambda i: (0, 0)
    return pl.pallas_call(
        _node_body,
        grid=grid,
        in_specs=[
            pl.BlockSpec((bn, D), lambda i: (i, 0)),
            pl.BlockSpec((bn, 4), lambda i: (i, 0)),
            pl.BlockSpec((2, bn, D), lambda i: (0, i, 0)),
            pl.BlockSpec((2, bn, 4), lambda i: (0, i, 0)),
            pl.BlockSpec((2, bn, D), lambda i: (0, i, 0)),
            pl.BlockSpec((2, bn, 4), lambda i: (0, i, 0)),
        ] + [pl.BlockSpec(w.shape, full) for w in wts],
        out_specs=[
            pl.BlockSpec((bn, D), lambda i: (i, 0)),
            pl.BlockSpec((bn, 4), lambda i: (i, 0)),
            pl.BlockSpec((bn, D), lambda i: (i, 0)),
            pl.BlockSpec((bn, D), lambda i: (i, 0)),
        ],
        out_shape=[
            jax.ShapeDtypeStruct((n, D), jnp.float32),
            jax.ShapeDtypeStruct((n, 4), jnp.float32),
            jax.ShapeDtypeStruct((n, D), jnp.float32),
            jax.ShapeDtypeStruct((n, D), jnp.float32),
        ],
    )(h, pos4, magg_p, pagg_p, magg_pb, pagg_pb, *wts)


def _final_body(pos_ref, pagg_ref, paggb_ref, posn_ref):
    pagg = pagg_ref[0] + pagg_ref[1] + paggb_ref[0] + paggb_ref[1]
    lanes = lax.broadcasted_iota(jnp.int32, (1, 4), 1)
    cnt = jnp.sum(pagg * (lanes == 3).astype(jnp.float32), axis=-1, keepdims=True)
    cnt = jnp.maximum(cnt, 1.0)
    posd = pagg * (lanes < 3).astype(jnp.float32) / cnt
    posn_ref[...] = pos_ref[...] + posd


def _tc_final(pos4, pagg_p, pagg_pb, n, bn):
    grid = (n // bn,)
    return pl.pallas_call(
        _final_body,
        grid=grid,
        in_specs=[
            pl.BlockSpec((bn, 4), lambda i: (i, 0)),
            pl.BlockSpec((2, bn, 4), lambda i: (0, i, 0)),
            pl.BlockSpec((2, bn, 4), lambda i: (0, i, 0)),
        ],
        out_specs=pl.BlockSpec((bn, 4), lambda i: (i, 0)),
        out_shape=jax.ShapeDtypeStruct((n, 4), jnp.float32),
    )(pos4, pagg_p, pagg_pb)


def kernel(atoms, pos, edge_index, params):
    n = atoms.shape[0]
    e = edge_index.shape[1]
    layers = params["layers"]

    emb = params["embedding"]
    emb_pad = jnp.zeros((D, D), jnp.float32).at[: emb.shape[0]].set(emb)

    def msg1_parts(lp):
        w1 = lp["msg_l1"]["W"]
        return (w1[:D], w1[D:2 * D], w1[2 * D:2 * D + 1],
                lp["msg_l1"]["b"].reshape(1, D))

    src = edge_index[0].astype(jnp.int32)
    dst = edge_index[1].astype(jnp.int32)
    src2 = src.reshape(-1, CHUNK)
    dst2 = dst.reshape(-1, CHUNK)
    posf = pos.astype(jnp.float32)
    pos4 = jnp.pad(posf, ((0, 0), (0, 4 - pos.shape[1])))

    bn = 2000
    be = 3200
    w1a0, w1b0, _, b10 = msg1_parts(layers[0])
    h, a_cur, b_cur = _tc_init(atoms.astype(jnp.int32), emb_pad,
                               w1a0, w1b0, b10, n, bn)

    np_ = ((n + NW * 8 - 1) // (NW * 8)) * NW * 8
    eh = e // 2
    gather_k = _make_sc_gather(n, eh)
    scatter_k = _make_sc_scatter(np_, eh)
    scatter_pos_k = _make_sc_scatter_pos(np_, eh)
    nrows = e // CHUNK
    src2h = [src2[: nrows // 2], src2[nrows // 2:]]
    dst2h = [dst2[: nrows // 2], dst2[nrows // 2:]]
    dsth = [dst[:eh], dst[eh:]]

    for li, lp in enumerate(layers):
        px, py, pz = pos4[:, 0], pos4[:, 1], pos4[:, 2]

        wd = msg1_parts(lp)[2]
        ewts = [
            wd,
            lp["msg_ln1"]["g"].reshape(1, D), lp["msg_ln1"]["b"].reshape(1, D),
            lp["msg_l2"]["W"], lp["msg_l2"]["b"].reshape(1, D),
            lp["msg_ln2"]["g"].reshape(1, D), lp["msg_ln2"]["b"].reshape(1, D),
            lp["pos_l1"]["W"], lp["pos_l1"]["b"].reshape(1, D),
            lp["pos_ln1"]["g"].reshape(1, D), lp["pos_ln1"]["b"].reshape(1, D),
            lp["pos_l2"]["W"].reshape(1, D), lp["pos_l2"]["b"].reshape(1, 1),
        ]
        gh = [gather_k(a_cur, b_cur, px, py, pz, src2h[hf], dst2h[hf])
              for hf in (0, 1)]
        last = li + 1 >= len(layers)
        if not last:
            eh_out = [_tc_edge(gh[hf][0], gh[hf][1], ewts, eh, be)
                      for hf in (0, 1)]
            sc = [scatter_k(eh_out[hf][0], eh_out[hf][1], gh[hf][1], dsth[hf])
                  for hf in (0, 1)]
            pagg_fl = [s[1] for s in sc]
        else:
            pwh = [_tc_edge_pw(gh[hf][0], gh[hf][1], ewts, eh, be)
                   for hf in (0, 1)]
            sc = [[None, None], [None, None]]
            pagg_fl = [scatter_pos_k(pwh[hf], gh[hf][1], dsth[hf])
                       for hf in (0, 1)]
            pagg_fl = [p[0] if isinstance(p, (tuple, list)) else p
                       for p in pagg_fl]
        pagg_ps = [p.reshape(2, 4, np_).transpose(0, 2, 1) for p in pagg_fl]

        if li + 1 < len(layers):
            w1a_n, w1b_n, _, b1_n = msg1_parts(layers[li + 1])
            wu1 = lp["upd_l1"]["W"]
            nwts = [
                wu1[:D], wu1[D:], lp["upd_l1"]["b"].reshape(1, D),
                lp["upd_ln1"]["g"].reshape(1, D), lp["upd_ln1"]["b"].reshape(1, D),
                lp["upd_l2"]["W"], lp["upd_l2"]["b"].reshape(1, D),
                lp["upd_ln2"]["g"].reshape(1, D), lp["upd_ln2"]["b"].reshape(1, D),
                w1a_n, w1b_n, b1_n,
            ]
            h, pos4, a_cur, b_cur = _tc_node(h, pos4, sc[0][0], pagg_ps[0],
                                             sc[1][0], pagg_ps[1], nwts, n, bn)
        else:
            pos4 = _tc_final(pos4, pagg_ps[0], pagg_ps[1], n, bn)

    return pos4[:, :3]
